# Initial kernel scaffold; baseline (speedup 1.0000x reference)
import jax
import jax.numpy as jnp
from jax.experimental import pallas as pl


def kernel(xyz, points, conv_w0, conv_b0, bn_g0, bn_b0, conv_w1, conv_b1, bn_g1, bn_b1):
    raise NotImplementedError("write your pallas kernel here")


# speedup vs baseline: 6.7894x; 6.7894x over previous
import functools

import jax
import jax.numpy as jnp
from jax import lax
from jax.experimental import pallas as pl
from jax.experimental.pallas import tpu as pltpu
from jax.experimental.pallas import tpu_sc as plsc

KNB = 32
EPSV = 1e-5
FPAD = 128
OC = 64


TOPK_R = 256


def _topk_body(rows_ref, xyzt_ref, out_ref):
    b = pl.program_id(0)
    x = rows_ref[0]
    xt = xyzt_ref[0]
    nr = jnp.sum(x * x, axis=1, keepdims=True)
    nc = jnp.sum(xt * xt, axis=0, keepdims=True)
    d = (nr + nc) - 2.0 * jnp.dot(x, xt, preferred_element_type=jnp.float32)
    r, n = d.shape
    col = lax.broadcasted_iota(jnp.int32, (r, n), 1)
    kcol = lax.broadcasted_iota(jnp.int32, (r, KNB), 1)
    big = jnp.int32(2 ** 30)
    inf = jnp.float32(jnp.inf)

    def body(k, carry):
        dd, acc = carry
        m = jnp.min(dd, axis=1, keepdims=True)
        eq = dd == m
        am = jnp.min(jnp.where(eq, col, big), axis=1, keepdims=True)
        dd = jnp.where(col == am, inf, dd)
        acc = jnp.where(kcol == k, am, acc)
        return dd, acc

    _, acc = lax.fori_loop(0, KNB, body, (d, jnp.zeros((r, KNB), jnp.int32)))
    out_ref[0] = acc + b * n


def _topk(xyz):
    bsz, n, _ = xyz.shape
    xyzt = jnp.transpose(xyz, (0, 2, 1))
    return pl.pallas_call(
        _topk_body,
        grid=(bsz, n // TOPK_R),
        in_specs=[
            pl.BlockSpec((1, TOPK_R, 3), lambda b, r: (b, r, 0)),
            pl.BlockSpec((1, 3, n), lambda b, r: (b, 0, 0)),
        ],
        out_specs=pl.BlockSpec((1, TOPK_R, KNB), lambda b, r: (b, r, 0)),
        out_shape=jax.ShapeDtypeStruct((bsz, n, KNB), jnp.int32),
    )(xyz, xyzt)


SC_NC = 2
SC_NS = 16
SC_NW = SC_NC * SC_NS
GCHUNK = 128


def _gather_rows(table, idx3, nrows):
    per_w = nrows // SC_NW
    nch = per_w // GCHUNK
    mesh = plsc.VectorSubcoreMesh(core_axis_name="c", subcore_axis_name="s")

    @functools.partial(
        pl.kernel,
        mesh=mesh,
        out_type=jax.ShapeDtypeStruct((nrows, FPAD), jnp.float32),
        scratch_types=[
            pltpu.VMEM((nch, GCHUNK), jnp.int32),
            pltpu.VMEM((GCHUNK, FPAD), jnp.float32),
            pltpu.SemaphoreType.DMA,
        ],
    )
    def gather_k(idx_hbm, table_hbm, out_hbm, idx_v, buf, sem):
        wid = lax.axis_index("s") * SC_NC + lax.axis_index("c")
        base = wid * per_w
        pltpu.sync_copy(idx_hbm.at[wid], idx_v)

        def body(c, carry):
            pltpu.async_copy(table_hbm.at[idx_v.at[c]], buf, sem).wait()
            pltpu.sync_copy(buf, out_hbm.at[pl.ds(base + c * GCHUNK, GCHUNK)])
            return carry

        lax.fori_loop(0, nch, body, 0)

    return gather_k(idx3, table)


TROWS = 2048
TPTS = TROWS // KNB


def _stats1_body(feat_ref, cx_ref, w0_ref, wx_ref, b0_ref, s_ref, ss_ref):
    i = pl.program_id(0)
    f = feat_ref[...]
    y = jnp.dot(f, w0_ref[...], preferred_element_type=jnp.float32)
    y = y + b0_ref[...]
    corr = jnp.dot(cx_ref[...], wx_ref[...],
                   preferred_element_type=jnp.float32)
    y3 = y.reshape(TPTS, KNB, OC) - corr[:, None, :]

    @pl.when(i == 0)
    def _init():
        s_ref[...] = jnp.zeros_like(s_ref)
        ss_ref[...] = jnp.zeros_like(ss_ref)

    s_ref[...] += jnp.sum(y3, axis=(0, 1))[None, :]
    ss_ref[...] += jnp.sum(y3 * y3, axis=(0, 1))[None, :]


def _stats1(feat, cx, w0p, w0x, b0):
    nrows = feat.shape[0]
    grid = (nrows // TROWS,)
    return pl.pallas_call(
        _stats1_body,
        grid=grid,
        in_specs=[
            pl.BlockSpec((TROWS, FPAD), lambda i: (i, 0)),
            pl.BlockSpec((TPTS, 3), lambda i: (i, 0)),
            pl.BlockSpec((FPAD, OC), lambda i: (0, 0)),
            pl.BlockSpec((3, OC), lambda i: (0, 0)),
            pl.BlockSpec((1, OC), lambda i: (0, 0)),
        ],
        out_specs=[
            pl.BlockSpec((1, OC), lambda i: (0, 0)),
            pl.BlockSpec((1, OC), lambda i: (0, 0)),
        ],
        out_shape=[
            jax.ShapeDtypeStruct((1, OC), jnp.float32),
            jax.ShapeDtypeStruct((1, OC), jnp.float32),
        ],
    )(feat, cx, w0p, w0x, b0)


def _pass2_body(feat_ref, cx_ref, w0_ref, wx_ref, b0_ref, sc1_ref, sh1_ref,
                w1_ref, b1_ref, s_ref, ss_ref, mx_ref, mn_ref):
    i = pl.program_id(0)
    f = feat_ref[...]
    y = jnp.dot(f, w0_ref[...], preferred_element_type=jnp.float32)
    y = y + b0_ref[...]
    corr = jnp.dot(cx_ref[...], wx_ref[...],
                   preferred_element_type=jnp.float32)
    y3 = y.reshape(TPTS, KNB, OC) - corr[:, None, :]
    z = jnp.maximum(y3 * sc1_ref[...].reshape(1, 1, OC)
                    + sh1_ref[...].reshape(1, 1, OC), 0.0)
    y2 = jnp.dot(z.reshape(TROWS, OC), w1_ref[...],
                 preferred_element_type=jnp.float32) + b1_ref[...]

    @pl.when(i == 0)
    def _init():
        s_ref[...] = jnp.zeros_like(s_ref)
        ss_ref[...] = jnp.zeros_like(ss_ref)

    s_ref[...] += jnp.sum(y2, axis=0, keepdims=True)
    ss_ref[...] += jnp.sum(y2 * y2, axis=0, keepdims=True)
    y23 = y2.reshape(TPTS, KNB, OC)
    mx_ref[...] = jnp.max(y23, axis=1)
    mn_ref[...] = jnp.min(y23, axis=1)


def _pass2(feat, cx, w0p, w0x, b0, sc1, sh1, w1t, b1):
    nrows = feat.shape[0]
    npts = nrows // KNB
    grid = (nrows // TROWS,)
    return pl.pallas_call(
        _pass2_body,
        grid=grid,
        in_specs=[
            pl.BlockSpec((TROWS, FPAD), lambda i: (i, 0)),
            pl.BlockSpec((TPTS, 3), lambda i: (i, 0)),
            pl.BlockSpec((FPAD, OC), lambda i: (0, 0)),
            pl.BlockSpec((3, OC), lambda i: (0, 0)),
            pl.BlockSpec((1, OC), lambda i: (0, 0)),
            pl.BlockSpec((1, OC), lambda i: (0, 0)),
            pl.BlockSpec((1, OC), lambda i: (0, 0)),
            pl.BlockSpec((OC, OC), lambda i: (0, 0)),
            pl.BlockSpec((1, OC), lambda i: (0, 0)),
        ],
        out_specs=[
            pl.BlockSpec((1, OC), lambda i: (0, 0)),
            pl.BlockSpec((1, OC), lambda i: (0, 0)),
            pl.BlockSpec((TPTS, OC), lambda i: (i, 0)),
            pl.BlockSpec((TPTS, OC), lambda i: (i, 0)),
        ],
        out_shape=[
            jax.ShapeDtypeStruct((1, OC), jnp.float32),
            jax.ShapeDtypeStruct((1, OC), jnp.float32),
            jax.ShapeDtypeStruct((npts, OC), jnp.float32),
            jax.ShapeDtypeStruct((npts, OC), jnp.float32),
        ],
    )(feat, cx, w0p, w0x, b0, sc1, sh1, w1t, b1)


def _final_body(mx_ref, mn_ref, sc_ref, sh_ref, o_ref):
    a = mx_ref[...] * sc_ref[...] + sh_ref[...]
    b = mn_ref[...] * sc_ref[...] + sh_ref[...]
    o_ref[...] = jnp.maximum(jnp.maximum(a, b), 0.0)


def _final(mx, mn, sc2, sh2):
    npts = mx.shape[0]
    return pl.pallas_call(
        _final_body,
        grid=(1,),
        in_specs=[
            pl.BlockSpec((npts, OC), lambda i: (0, 0)),
            pl.BlockSpec((npts, OC), lambda i: (0, 0)),
            pl.BlockSpec((1, OC), lambda i: (0, 0)),
            pl.BlockSpec((1, OC), lambda i: (0, 0)),
        ],
        out_specs=pl.BlockSpec((npts, OC), lambda i: (0, 0)),
        out_shape=jax.ShapeDtypeStruct((npts, OC), jnp.float32),
    )(mx, mn, sc2, sh2)


def kernel(xyz, points, conv_w0, conv_b0, bn_g0, bn_b0,
           conv_w1, conv_b1, bn_g1, bn_b1):
    bsz, n, _ = xyz.shape
    d = points.shape[-1]
    bn = bsz * n
    nrows = bn * KNB

    idx = _topk(xyz)

    table = jnp.concatenate([xyz, points], axis=-1).reshape(bn, 3 + d)
    table = jnp.pad(table, ((0, 0), (0, FPAD - (3 + d))))
    idx3 = idx.reshape(SC_NW, nrows // (SC_NW * GCHUNK), GCHUNK)
    feat = _gather_rows(table, idx3, nrows)

    w0p = jnp.pad(jnp.transpose(conv_w0), ((0, FPAD - (3 + d)), (0, 0)))
    w0x = jnp.transpose(conv_w0[:, :3])
    b0 = conv_b0.reshape(1, OC)
    cx = xyz.reshape(bn, 3)

    cnt = float(nrows)
    s1, ss1 = _stats1(feat, cx, w0p, w0x, b0)
    mean1 = s1 / cnt
    var1 = ss1 / cnt - mean1 * mean1
    sc1 = bn_g0.reshape(1, OC) / jnp.sqrt(var1 + EPSV)
    sh1 = bn_b0.reshape(1, OC) - mean1 * sc1

    s2, ss2, mx, mn = _pass2(feat, cx, w0p, w0x, b0, sc1, sh1,
                             jnp.transpose(conv_w1), conv_b1.reshape(1, OC))
    mean2 = s2 / cnt
    var2 = ss2 / cnt - mean2 * mean2
    sc2 = bn_g1.reshape(1, OC) / jnp.sqrt(var2 + EPSV)
    sh2 = bn_b1.reshape(1, OC) - mean2 * sc2

    out = _final(mx, mn, sc2, sh2)
    return out.reshape(bsz, n, OC)
